# Initial kernel scaffold; baseline (speedup 1.0000x reference)
import functools

import jax
import jax.numpy as jnp
from jax import lax
from jax.experimental import pallas as pl
from jax.experimental.pallas import tpu as pltpu
from jax.experimental.pallas import tpu_sc as plsc

L = 4096
TOTAL = L * L
NUM_WORKERS = 32
PER_WORKER = TOTAL // NUM_WORKERS
CHUNK = 16384
NUM_CHUNKS = PER_WORKER // CHUNK
VECS_PER_CHUNK = CHUNK // 16


def _sc_lookup(table16, idx_flat):
    mesh = plsc.VectorSubcoreMesh(core_axis_name="c", subcore_axis_name="s")

    @functools.partial(
        pl.kernel,
        mesh=mesh,
        out_type=jax.ShapeDtypeStruct((TOTAL,), jnp.float32),
        scratch_types=[
            pltpu.VMEM((16,), jnp.float32),
            pltpu.VMEM((CHUNK,), jnp.int32),
            pltpu.VMEM((CHUNK,), jnp.float32),
        ],
    )
    def k(table_hbm, idx_hbm, out_hbm, tab_v, idx_v, out_v):
        wid = lax.axis_index("s") * 2 + lax.axis_index("c")
        base = wid * PER_WORKER
        pltpu.sync_copy(table_hbm, tab_v)

        def chunk_body(ci, carry):
            off = base + ci * CHUNK
            pltpu.sync_copy(idx_hbm.at[pl.ds(off, CHUNK)], idx_v)

            def vec_body(vi, c2):
                iv = idx_v[pl.ds(vi * 16, 16)]
                out_v[pl.ds(vi * 16, 16)] = plsc.load_gather(tab_v, [iv])
                return c2

            lax.fori_loop(0, VECS_PER_CHUNK, vec_body, 0, unroll=8)
            pltpu.sync_copy(out_v, out_hbm.at[pl.ds(off, CHUNK)])
            return carry

        lax.fori_loop(0, NUM_CHUNKS, chunk_body, 0)

    return k(table16, idx_flat)


def kernel(selected_ids, crf_transitions_model):
    idx_flat = selected_ids.reshape(-1).astype(jnp.int32)
    flat = crf_transitions_model.reshape(-1)
    table16 = jnp.concatenate([flat, jnp.zeros((1,), jnp.float32)])
    out = _sc_lookup(table16, idx_flat)
    return out.reshape(selected_ids.shape)


# speedup vs baseline: 4.5750x; 4.5750x over previous
import functools

import jax
import jax.numpy as jnp
from jax import lax
from jax.experimental import pallas as pl
from jax.experimental.pallas import tpu as pltpu
from jax.experimental.pallas import tpu_sc as plsc

L = 4096
TOTAL = L * L
NUM_WORKERS = 32
PER_WORKER = TOTAL // NUM_WORKERS
CHUNK = 16384
NUM_CHUNKS = PER_WORKER // CHUNK
VECS_PER_CHUNK = CHUNK // 16


def _sc_lookup(table16, idx_flat):
    mesh = plsc.VectorSubcoreMesh(core_axis_name="c", subcore_axis_name="s")

    @functools.partial(
        pl.kernel,
        mesh=mesh,
        out_type=jax.ShapeDtypeStruct((TOTAL,), jnp.float32),
        compiler_params=pltpu.CompilerParams(needs_layout_passes=False),
        scratch_types=[
            pltpu.VMEM((16,), jnp.float32),
            pltpu.VMEM((CHUNK,), jnp.int32),
            pltpu.VMEM((CHUNK,), jnp.float32),
        ],
    )
    def k(table_hbm, idx_hbm, out_hbm, tab_v, idx_v, out_v):
        wid = lax.axis_index("s") * 2 + lax.axis_index("c")
        base = wid * PER_WORKER
        pltpu.sync_copy(table_hbm, tab_v)

        def chunk_body(ci, carry):
            off = base + ci * CHUNK
            pltpu.sync_copy(idx_hbm.at[pl.ds(off, CHUNK)], idx_v)

            def vec_body(vi, c2):
                iv = idx_v[pl.ds(vi * 16, 16)]
                out_v[pl.ds(vi * 16, 16)] = plsc.load_gather(tab_v, [iv])
                return c2

            lax.fori_loop(0, VECS_PER_CHUNK, vec_body, 0, unroll=8)
            pltpu.sync_copy(out_v, out_hbm.at[pl.ds(off, CHUNK)])
            return carry

        lax.fori_loop(0, NUM_CHUNKS, chunk_body, 0)

    return k(table16, idx_flat)


def kernel(selected_ids, crf_transitions_model):
    idx_flat = selected_ids.reshape(-1).astype(jnp.int32)
    flat = crf_transitions_model.reshape(-1)
    table16 = jnp.concatenate([flat, jnp.zeros((1,), jnp.float32)])
    out = _sc_lookup(table16, idx_flat)
    return out.reshape(selected_ids.shape)


# speedup vs baseline: 11.2654x; 2.4624x over previous
import functools

import jax
import jax.numpy as jnp
from jax import lax
from jax.experimental import pallas as pl
from jax.experimental.pallas import tpu as pltpu
from jax.experimental.pallas import tpu_sc as plsc

L = 4096
TOTAL = L * L
NUM_WORKERS = 32
PER_WORKER = TOTAL // NUM_WORKERS
CHUNK = 16384
NUM_CHUNKS = PER_WORKER // CHUNK
VECS_PER_CHUNK = CHUNK // 16
NBUF = 2
ROUNDS = NUM_CHUNKS // NBUF


def _sc_lookup(table16, idx_flat):
    mesh = plsc.VectorSubcoreMesh(core_axis_name="c", subcore_axis_name="s")

    @functools.partial(
        pl.kernel,
        mesh=mesh,
        out_type=jax.ShapeDtypeStruct((TOTAL,), jnp.float32),
        compiler_params=pltpu.CompilerParams(needs_layout_passes=False),
        scratch_types=[
            pltpu.VMEM((16,), jnp.float32),
            pltpu.VMEM((CHUNK,), jnp.int32),
            pltpu.VMEM((CHUNK,), jnp.int32),
            pltpu.VMEM((CHUNK,), jnp.float32),
            pltpu.VMEM((CHUNK,), jnp.float32),
            pltpu.SemaphoreType.DMA,
            pltpu.SemaphoreType.DMA,
            pltpu.SemaphoreType.DMA,
            pltpu.SemaphoreType.DMA,
        ],
    )
    def k(table_hbm, idx_hbm, out_hbm, tab_v, idx0, idx1, o0, o1,
          si0, si1, so0, so1):
        wid = lax.axis_index("s") * 2 + lax.axis_index("c")
        base = wid * PER_WORKER
        pltpu.sync_copy(table_hbm, tab_v)
        idx_b = (idx0, idx1)
        out_b = (o0, o1)
        sin = (si0, si1)
        sout = (so0, so1)

        for b in range(NBUF):
            pltpu.async_copy(
                idx_hbm.at[pl.ds(base + b * CHUNK, CHUNK)], idx_b[b], sin[b])

        def round_body(r, carry):
            for b in range(NBUF):
                ci = r * NBUF + b
                off = base + ci * CHUNK
                pltpu.make_async_copy(
                    idx_hbm.at[pl.ds(off, CHUNK)], idx_b[b], sin[b]).wait()

                @pl.when(r > 0)
                def _wait_out():
                    pltpu.make_async_copy(
                        out_b[b], out_hbm.at[pl.ds(off, CHUNK)], sout[b]).wait()

                @plsc.parallel_loop(0, VECS_PER_CHUNK, 1, unroll=8)
                def _vec(vi):
                    s = pl.ds(vi * 16, 16)
                    out_b[b][s] = plsc.load_gather(tab_v, [idx_b[b][s]])

                pltpu.async_copy(
                    out_b[b], out_hbm.at[pl.ds(off, CHUNK)], sout[b])

                @pl.when(ci + NBUF < NUM_CHUNKS)
                def _prefetch():
                    off2 = base + (ci + NBUF) * CHUNK
                    pltpu.async_copy(
                        idx_hbm.at[pl.ds(off2, CHUNK)], idx_b[b], sin[b])

            return carry

        lax.fori_loop(0, ROUNDS, round_body, 0)

        for b in range(NBUF):
            pltpu.make_async_copy(
                out_b[b], out_hbm.at[pl.ds(base, CHUNK)], sout[b]).wait()

    return k(table16, idx_flat)


def kernel(selected_ids, crf_transitions_model):
    idx_flat = selected_ids.reshape(-1).astype(jnp.int32)
    flat = crf_transitions_model.reshape(-1)
    table16 = jnp.concatenate([flat, jnp.zeros((1,), jnp.float32)])
    out = _sc_lookup(table16, idx_flat)
    return out.reshape(selected_ids.shape)


# speedup vs baseline: 11.6017x; 1.0299x over previous
import functools

import jax
import jax.numpy as jnp
from jax import lax
from jax.experimental import pallas as pl
from jax.experimental.pallas import tpu as pltpu
from jax.experimental.pallas import tpu_sc as plsc

L = 4096
TOTAL = L * L
NUM_WORKERS = 32
PER_WORKER = TOTAL // NUM_WORKERS
CHUNK = 16384
NUM_CHUNKS = PER_WORKER // CHUNK
VECS_PER_CHUNK = CHUNK // 16
NBUF = 2
ROUNDS = NUM_CHUNKS // NBUF


def _sc_lookup(table16, idx_flat):
    mesh = plsc.VectorSubcoreMesh(core_axis_name="c", subcore_axis_name="s")

    @functools.partial(
        pl.kernel,
        mesh=mesh,
        out_type=jax.ShapeDtypeStruct((TOTAL,), jnp.float32),
        compiler_params=pltpu.CompilerParams(needs_layout_passes=False),
        scratch_types=[
            pltpu.VMEM((16,), jnp.float32),
            pltpu.VMEM((CHUNK,), jnp.int32),
            pltpu.VMEM((CHUNK,), jnp.int32),
            pltpu.VMEM((CHUNK,), jnp.float32),
            pltpu.VMEM((CHUNK,), jnp.float32),
            pltpu.SemaphoreType.DMA,
            pltpu.SemaphoreType.DMA,
            pltpu.SemaphoreType.DMA,
            pltpu.SemaphoreType.DMA,
        ],
    )
    def k(table_hbm, idx_hbm, out_hbm, tab_v, idx0, idx1, o0, o1,
          si0, si1, so0, so1):
        wid = lax.axis_index("s") * 2 + lax.axis_index("c")
        base = wid * PER_WORKER
        pltpu.sync_copy(table_hbm, tab_v)
        tab_vec = tab_v[...]
        idx_b = (idx0, idx1)
        out_b = (o0, o1)
        sin = (si0, si1)
        sout = (so0, so1)

        for b in range(NBUF):
            pltpu.async_copy(
                idx_hbm.at[pl.ds(base + b * CHUNK, CHUNK)], idx_b[b], sin[b])

        def round_body(r, carry):
            for b in range(NBUF):
                ci = r * NBUF + b
                off = base + ci * CHUNK
                pltpu.make_async_copy(
                    idx_hbm.at[pl.ds(off, CHUNK)], idx_b[b], sin[b]).wait()

                @pl.when(r > 0)
                def _wait_out():
                    pltpu.make_async_copy(
                        out_b[b], out_hbm.at[pl.ds(off, CHUNK)], sout[b]).wait()

                @plsc.parallel_loop(0, VECS_PER_CHUNK, 1, unroll=8)
                def _vec(vi):
                    s = pl.ds(vi * 16, 16)
                    out_b[b][s] = jnp.take_along_axis(
                        tab_vec, idx_b[b][s], axis=0,
                        mode="promise_in_bounds")

                pltpu.async_copy(
                    out_b[b], out_hbm.at[pl.ds(off, CHUNK)], sout[b])

                @pl.when(ci + NBUF < NUM_CHUNKS)
                def _prefetch():
                    off2 = base + (ci + NBUF) * CHUNK
                    pltpu.async_copy(
                        idx_hbm.at[pl.ds(off2, CHUNK)], idx_b[b], sin[b])

            return carry

        lax.fori_loop(0, ROUNDS, round_body, 0)

        for b in range(NBUF):
            pltpu.make_async_copy(
                out_b[b], out_hbm.at[pl.ds(base, CHUNK)], sout[b]).wait()

    return k(table16, idx_flat)


def kernel(selected_ids, crf_transitions_model):
    idx_flat = selected_ids.reshape(-1).astype(jnp.int32)
    flat = crf_transitions_model.reshape(-1)
    table16 = jnp.concatenate([flat, jnp.zeros((1,), jnp.float32)])
    out = _sc_lookup(table16, idx_flat)
    return out.reshape(selected_ids.shape)


# speedup vs baseline: 29.5193x; 2.5444x over previous
import functools

import jax
import jax.numpy as jnp
from jax import lax
from jax.experimental import pallas as pl
from jax.experimental.pallas import tpu as pltpu
from jax.experimental.pallas import tpu_sc as plsc

L = 4096
NUM_WORKERS = 32
ROWS_PER_WORKER = L // NUM_WORKERS
ROWS_PER_CHUNK = 4
NUM_CHUNKS = ROWS_PER_WORKER // ROWS_PER_CHUNK
VECS_PER_ROW = L // 16
NBUF = 2
ROUNDS = NUM_CHUNKS // NBUF


def _sc_lookup(table16, idx):
    mesh = plsc.VectorSubcoreMesh(core_axis_name="c", subcore_axis_name="s")

    @functools.partial(
        pl.kernel,
        mesh=mesh,
        out_type=jax.ShapeDtypeStruct((L, L), jnp.float32),
        compiler_params=pltpu.CompilerParams(needs_layout_passes=False),
        scratch_types=[
            pltpu.VMEM((16,), jnp.float32),
            pltpu.VMEM((ROWS_PER_CHUNK, L), jnp.int32),
            pltpu.VMEM((ROWS_PER_CHUNK, L), jnp.int32),
            pltpu.VMEM((ROWS_PER_CHUNK, L), jnp.float32),
            pltpu.VMEM((ROWS_PER_CHUNK, L), jnp.float32),
            pltpu.SemaphoreType.DMA,
            pltpu.SemaphoreType.DMA,
            pltpu.SemaphoreType.DMA,
            pltpu.SemaphoreType.DMA,
        ],
    )
    def k(table_hbm, idx_hbm, out_hbm, tab_v, idx0, idx1, o0, o1,
          si0, si1, so0, so1):
        wid = lax.axis_index("s") * 2 + lax.axis_index("c")
        row_base = wid * ROWS_PER_WORKER
        pltpu.sync_copy(table_hbm, tab_v)
        tab_vec = tab_v[...]
        idx_b = (idx0, idx1)
        out_b = (o0, o1)
        sin = (si0, si1)
        sout = (so0, so1)

        for b in range(NBUF):
            pltpu.async_copy(
                idx_hbm.at[pl.ds(row_base + b * ROWS_PER_CHUNK,
                                 ROWS_PER_CHUNK)],
                idx_b[b], sin[b])

        def round_body(r, carry):
            for b in range(NBUF):
                ci = r * NBUF + b
                r0 = row_base + ci * ROWS_PER_CHUNK
                pltpu.make_async_copy(
                    idx_hbm.at[pl.ds(r0, ROWS_PER_CHUNK)],
                    idx_b[b], sin[b]).wait()

                @pl.when(r > 0)
                def _wait_out():
                    pltpu.make_async_copy(
                        out_b[b], out_hbm.at[pl.ds(r0, ROWS_PER_CHUNK)],
                        sout[b]).wait()

                for row in range(ROWS_PER_CHUNK):
                    @plsc.parallel_loop(0, VECS_PER_ROW, 1, unroll=8)
                    def _vec(vi):
                        s = pl.ds(vi * 16, 16)
                        out_b[b][row, s] = plsc.load_gather(
                            tab_v, [idx_b[b][row, s]])

                pltpu.async_copy(
                    out_b[b], out_hbm.at[pl.ds(r0, ROWS_PER_CHUNK)], sout[b])

                @pl.when(ci + NBUF < NUM_CHUNKS)
                def _prefetch():
                    r2 = row_base + (ci + NBUF) * ROWS_PER_CHUNK
                    pltpu.async_copy(
                        idx_hbm.at[pl.ds(r2, ROWS_PER_CHUNK)],
                        idx_b[b], sin[b])

            return carry

        lax.fori_loop(0, ROUNDS, round_body, 0)

        for b in range(NBUF):
            pltpu.make_async_copy(
                out_b[b], out_hbm.at[pl.ds(row_base, ROWS_PER_CHUNK)],
                sout[b]).wait()

    return k(table16, idx)


def kernel(selected_ids, crf_transitions_model):
    idx = selected_ids.astype(jnp.int32)
    flat = crf_transitions_model.reshape(-1)
    table16 = jnp.concatenate([flat, jnp.zeros((1,), jnp.float32)])
    return _sc_lookup(table16, idx)
